# baseline (speedup 1.0000x reference)
import functools
import math

import jax
import jax.numpy as jnp
from jax import lax
from jax.experimental import pallas as pl
from jax.experimental.pallas import tpu as pltpu
from jax.experimental.pallas import tpu_sc as plsc


def _sc_kernel(B, F, h, w):
    hw = h * w
    C = 2 * F
    NC, NS, L = 2, 16, 16
    c_per = F // NS
    assert c_per * NS == F and w % L == 0 and h == w
    tab_words = F * h
    buf_words = c_per * hw
    mesh = plsc.VectorSubcoreMesh(core_axis_name="c", subcore_axis_name="s")

    @functools.partial(
        pl.kernel,
        mesh=mesh,
        out_type=jax.ShapeDtypeStruct((B, C * hw), jnp.float32),
        scratch_types=[
            pltpu.VMEM((c_per * w,), jnp.float32),
            pltpu.VMEM((buf_words,), jnp.float32),
            pltpu.SemaphoreType.DMA,
        ],
    )
    def k(rowT_hbm, colT_hbm, out_hbm, tab_v, buf_v, sem):
        core = lax.axis_index("c")
        sub = lax.axis_index("s")
        t_lo = sub * (c_per * w)

        @pl.when(core == 0)
        def _():
            pltpu.sync_copy(colT_hbm.at[pl.ds(t_lo, c_per * w)],
                            tab_v.at[pl.ds(0, c_per * w)])

        @pl.when(core == 1)
        def _():
            pltpu.sync_copy(rowT_hbm.at[pl.ds(t_lo, c_per * w)],
                            tab_v.at[pl.ds(0, c_per * w)])

        n_x_chunks = w // L

        @pl.when(core == 0)
        def _():
            for j in range(c_per):
                chunks = [tab_v[pl.ds(j * w + q * L, L)] for q in range(n_x_chunks)]

                def yloop(y, _, j=j, chunks=chunks):
                    base = j * hw + y * w
                    for q in range(n_x_chunks):
                        buf_v[pl.ds(base + q * L, L)] = chunks[q]
                    return 0

                lax.fori_loop(0, h, yloop, 0)

        @pl.when(core == 1)
        def _():
            for j in range(c_per):
                yvecs = [tab_v[pl.ds(j * w + q * L, L)] for q in range(h // L)]
                for y in range(h):
                    val = jnp.full((L,), yvecs[y // L][y % L], jnp.float32)
                    base = j * hw + y * w
                    for q in range(n_x_chunks):
                        buf_v[pl.ds(base + q * L, L)] = val

        c_lo = (core * F + sub * c_per) * hw
        KW = 8
        for wave in range(B // KW):
            for t in range(KW):
                b = wave * KW + t
                pltpu.make_async_copy(
                    buf_v, out_hbm.at[b, pl.ds(c_lo, buf_words)], sem
                ).start()
            for t in range(KW):
                pltpu.make_async_copy(
                    buf_v, out_hbm.at[wave * KW + t, pl.ds(c_lo, buf_words)], sem
                ).wait()

    return k


def kernel(token_tensors, row_embed_w, col_embed_w):
    B, _, h, w = token_tensors.shape
    F = row_embed_w.shape[1]
    rowT = row_embed_w.T.reshape(-1)
    colT = col_embed_w.T.reshape(-1)
    out = _sc_kernel(B, F, h, w)(rowT, colT)
    return out.reshape(B, 2 * F, h, w)
